# baseline (speedup 1.0000x reference)
import functools

import jax
import jax.numpy as jnp
from jax.experimental import pallas as pl

N_ATOM, N_RES, N_POS = 55, 21, 24
RES_OFF, POS_OFF = 64, 96
B1_ROW = 120
C = 128
H = 3 * C
ROW_BLOCK = 128


def _prep_body(atom_ref, res_ref, pos_ref, w1_ref, b1_ref, w2_ref,
               comb_ref, w2h_ref, sel_ref, maskj_ref):
    t, r = sel_ref.shape
    l = maskj_ref.shape[1]
    tt = jax.lax.broadcasted_iota(jnp.int32, (t, r), 0)
    rr = jax.lax.broadcasted_iota(jnp.int32, (t, r), 1)
    d = tt - l * rr
    sel_ref[:] = ((d >= 0) & (d < l)).astype(jnp.bfloat16)
    tj = jax.lax.broadcasted_iota(
        jnp.int32, (t, l), 0).astype(jnp.float32)
    jj = jax.lax.broadcasted_iota(
        jnp.int32, (t, l), 1).astype(jnp.float32)
    jm = tj - l * jnp.floor((tj + 0.5) * (1.0 / l))
    maskj_ref[:] = (jm == jj).astype(jnp.bfloat16)
    ca = jnp.dot(atom_ref[:], w1_ref[0:C, :],
                 preferred_element_type=jnp.float32)
    cr = jnp.dot(res_ref[:], w1_ref[C:2 * C, :],
                 preferred_element_type=jnp.float32)
    cp = jnp.dot(pos_ref[:], w1_ref[2 * C:3 * C, :],
                 preferred_element_type=jnp.float32)
    z = lambda k: jnp.zeros((k, H), dtype=jnp.float32)
    pieces = [ca, z(RES_OFF - N_ATOM), cr, z(POS_OFF - RES_OFF - N_RES), cp,
              z(B1_ROW - POS_OFF - N_POS), b1_ref[:], z(C - B1_ROW - 1)]
    comb = jnp.concatenate([p for p in pieces if p.shape[0] > 0], axis=0)
    comb_ref[:] = comb.astype(jnp.bfloat16)
    w2h_ref[:] = (w2_ref[:] * 0.5).astype(jnp.bfloat16)


def _bcast(sel, mask_j, ones_c, idx_ref):
    v = idx_ref[:].astype(jnp.float32).astype(jnp.bfloat16)
    y = jnp.dot(sel, v, preferred_element_type=jnp.float32)
    return jnp.dot(y.astype(jnp.bfloat16) * mask_j, ones_c,
                   preferred_element_type=jnp.float32)


def _main_body(atom_ref, res_ref, pos_ref, sel_ref, maskj_ref,
               comb_ref, w2h_ref, b2_ref, out_ref):
    r, l, _ = out_ref.shape
    t = r * l
    sel = sel_ref[:]
    mask_j = maskj_ref[:]
    ones_c = jnp.ones((l, C), dtype=jnp.bfloat16)
    ab = _bcast(sel, mask_j, ones_c, atom_ref)
    rb = _bcast(sel, mask_j, ones_c, res_ref)
    pb = _bcast(sel, mask_j, ones_c, pos_ref)
    iota = jax.lax.broadcasted_iota(
        jnp.int32, (t, C), 1).astype(jnp.float32)
    hit = ((iota == ab)
           | (iota == rb + 64.0)
           | (iota == pb + 96.0)
           | (iota == float(B1_ROW)))
    oh = hit.astype(jnp.bfloat16)
    h = jnp.dot(oh, comb_ref[:], preferred_element_type=jnp.float32)
    g = h * (1.0 + jax.lax.erf(h * 0.7071067811865476))
    g2 = jnp.dot(g.astype(jnp.bfloat16), w2h_ref[:],
                 preferred_element_type=jnp.float32) + b2_ref[:]
    for k in range(r):
        out_ref[k] = g2[k * l:(k + 1) * l, :]


@functools.partial(jax.jit, static_argnames=())
def kernel(atom_type, aa_type, aa_pos, atom_table, residue_table, pos_table,
           W1, b1, W2, b2):
    b, l = atom_type.shape

    r = ROW_BLOCK
    t = r * l
    grid = (b // r,)

    combined, w2_half, sel, mask_j = pl.pallas_call(
        _prep_body,
        out_shape=(jax.ShapeDtypeStruct((C, H), jnp.bfloat16),
                   jax.ShapeDtypeStruct((H, C), jnp.bfloat16),
                   jax.ShapeDtypeStruct((t, r), jnp.bfloat16),
                   jax.ShapeDtypeStruct((t, l), jnp.bfloat16)),
    )(atom_table, residue_table, pos_table, W1, b1.reshape(1, H), W2)

    idx_spec = pl.BlockSpec((r, l), lambda i: (i, 0))
    full = lambda shape: pl.BlockSpec(shape, lambda i: (0, 0))

    out = pl.pallas_call(
        _main_body,
        grid=grid,
        in_specs=[
            idx_spec, idx_spec, idx_spec,
            full((t, r)),
            full((t, l)),
            full((C, H)),
            full((H, C)),
            full((1, C)),
        ],
        out_specs=pl.BlockSpec((r, l, C), lambda i: (i, 0, 0)),
        out_shape=jax.ShapeDtypeStruct((b, l, C), jnp.float32),
    )(atom_type, aa_type, aa_pos, sel, mask_j,
      combined, w2_half, b2.reshape(1, C))

    return out


# speedup vs baseline: 1.3832x; 1.3832x over previous
import functools

import jax
import jax.numpy as jnp
from jax.experimental import pallas as pl

N_ATOM, N_RES, N_POS = 55, 21, 24
RES_OFF, POS_OFF = 64, 96
B1_ROW = 120
C = 128
H = 3 * C
ROW_BLOCK = 128


def _prep_body(atom_ref, res_ref, pos_ref, w1_ref, b1_ref, w2_ref,
               comb_ref, w2h_ref, selj_ref, maskr_ref):
    t, l = selj_ref.shape
    r = maskr_ref.shape[1]
    ca = jnp.dot(atom_ref[:], w1_ref[0:C, :],
                 preferred_element_type=jnp.float32)
    cr = jnp.dot(res_ref[:], w1_ref[C:2 * C, :],
                 preferred_element_type=jnp.float32)
    cp = jnp.dot(pos_ref[:], w1_ref[2 * C:3 * C, :],
                 preferred_element_type=jnp.float32)
    z = lambda k: jnp.zeros((k, H), dtype=jnp.float32)
    pieces = [ca, z(RES_OFF - N_ATOM), cr, z(POS_OFF - RES_OFF - N_RES), cp,
              z(B1_ROW - POS_OFF - N_POS), b1_ref[:], z(C - B1_ROW - 1)]
    comb = jnp.concatenate([p for p in pieces if p.shape[0] > 0], axis=0)
    comb_ref[:] = comb.astype(jnp.bfloat16)
    w2h_ref[:] = (w2_ref[:] * 0.5).astype(jnp.bfloat16)
    tt1 = jax.lax.broadcasted_iota(jnp.int32, (t, l), 0)
    jj = jax.lax.broadcasted_iota(jnp.int32, (t, l), 1)
    selj_ref[:] = (jax.lax.shift_right_logical(tt1, 7) == jj).astype(
        jnp.bfloat16)
    tt2 = jax.lax.broadcasted_iota(jnp.int32, (t, r), 0)
    rr = jax.lax.broadcasted_iota(jnp.int32, (t, r), 1)
    maskr_ref[:] = (jnp.bitwise_and(tt2, 127) == rr).astype(jnp.bfloat16)


def _bcast(selj, maskr, ones_c, idx_ref):
    v = idx_ref[:].astype(jnp.float32).astype(jnp.bfloat16)
    y = jnp.dot(selj, v, preferred_element_type=jnp.float32)
    return jnp.dot(y.astype(jnp.bfloat16) * maskr, ones_c,
                   preferred_element_type=jnp.float32)


def _main_body(atom_ref, res_ref, pos_ref, selj_ref, maskr_ref,
               comb_ref, w2h_ref, b2_ref, out_ref):
    l, r, _ = out_ref.shape
    t = l * r
    selj = selj_ref[:]
    maskr = maskr_ref[:]
    ones_c = jnp.ones((r, C), dtype=jnp.bfloat16)
    ab = _bcast(selj, maskr, ones_c, atom_ref)
    rb = _bcast(selj, maskr, ones_c, res_ref)
    pb = _bcast(selj, maskr, ones_c, pos_ref)
    iota = jax.lax.broadcasted_iota(
        jnp.int32, (t, C), 1).astype(jnp.float32)
    hit = ((iota == ab)
           | (iota == rb + float(RES_OFF))
           | (iota == pb + float(POS_OFF))
           | (iota == float(B1_ROW)))
    oh = hit.astype(jnp.bfloat16)
    h = jnp.dot(oh, comb_ref[:], preferred_element_type=jnp.float32)
    g = h * (1.0 + jax.lax.erf(h * 0.7071067811865476))
    g2 = jnp.dot(g.astype(jnp.bfloat16), w2h_ref[:],
                 preferred_element_type=jnp.float32) + b2_ref[:]
    out_ref[:] = g2.reshape(l, r, C)


@functools.partial(jax.jit, static_argnames=())
def kernel(atom_type, aa_type, aa_pos, atom_table, residue_table, pos_table,
           W1, b1, W2, b2):
    b, l = atom_type.shape
    r = ROW_BLOCK
    t = r * l
    grid = (b // r,)

    combined, w2_half, selj, maskr = pl.pallas_call(
        _prep_body,
        out_shape=(jax.ShapeDtypeStruct((C, H), jnp.bfloat16),
                   jax.ShapeDtypeStruct((H, C), jnp.bfloat16),
                   jax.ShapeDtypeStruct((t, l), jnp.bfloat16),
                   jax.ShapeDtypeStruct((t, r), jnp.bfloat16)),
    )(atom_table, residue_table, pos_table, W1, b1.reshape(1, H), W2)

    idx_spec = pl.BlockSpec((l, r), lambda i: (0, i))
    full = lambda shape: pl.BlockSpec(shape, lambda i: (0, 0))

    out = pl.pallas_call(
        _main_body,
        grid=grid,
        in_specs=[
            idx_spec, idx_spec, idx_spec,
            full((t, l)),
            full((t, r)),
            full((C, H)),
            full((H, C)),
            full((1, C)),
        ],
        out_specs=pl.BlockSpec((l, r, C), lambda i: (0, i, 0)),
        out_shape=jax.ShapeDtypeStruct((l, b, C), jnp.float32),
    )(atom_type.T, aa_type.T, aa_pos.T, selj, maskr,
      combined, w2_half, b2.reshape(1, C))

    return out.transpose(1, 0, 2)


# speedup vs baseline: 1.4802x; 1.0701x over previous
import functools

import jax
import jax.numpy as jnp
from jax.experimental import pallas as pl

N_ATOM, N_RES, N_POS = 55, 21, 24
RES_OFF, POS_OFF = 64, 96
B1_ROW = 120
C = 128
H = 3 * C
ROW_BLOCK = 128


def _prep_body(atom_ref, res_ref, pos_ref, w1_ref, b1_ref, w2_ref,
               comb_ref, w2h_ref, selj_ref, maskr_ref, bsel_ref):
    t, l = selj_ref.shape
    r = maskr_ref.shape[1]
    ca = jnp.dot(atom_ref[:], w1_ref[0:C, :],
                 preferred_element_type=jnp.float32)
    cr = jnp.dot(res_ref[:], w1_ref[C:2 * C, :],
                 preferred_element_type=jnp.float32)
    cp = jnp.dot(pos_ref[:], w1_ref[2 * C:3 * C, :],
                 preferred_element_type=jnp.float32)
    z = lambda k: jnp.zeros((k, H), dtype=jnp.float32)
    pieces = [ca, z(RES_OFF - N_ATOM), cr, z(POS_OFF - RES_OFF - N_RES), cp,
              z(B1_ROW - POS_OFF - N_POS), b1_ref[:], z(C - B1_ROW - 1)]
    comb = jnp.concatenate([p for p in pieces if p.shape[0] > 0], axis=0)
    comb_ref[:] = comb.astype(jnp.bfloat16)
    w2h_ref[:] = (w2_ref[:] * 0.5).astype(jnp.bfloat16)
    tt1 = jax.lax.broadcasted_iota(jnp.int32, (t, l), 0)
    jj = jax.lax.broadcasted_iota(jnp.int32, (t, l), 1)
    selj_ref[:] = (jax.lax.shift_right_logical(tt1, 7) == jj).astype(
        jnp.bfloat16)
    tt2 = jax.lax.broadcasted_iota(jnp.int32, (t, r), 0)
    rr = jax.lax.broadcasted_iota(jnp.int32, (t, r), 1)
    maskr_ref[:] = (jnp.bitwise_and(tt2, 127)
                    == jnp.bitwise_and(rr, 127)).astype(jnp.bfloat16)
    kk = jax.lax.broadcasted_iota(jnp.int32, (H, C), 0)
    cc = jax.lax.broadcasted_iota(jnp.int32, (H, C), 1)
    bsel_ref[:] = (((kk < C) & (cc < RES_OFF))
                   | ((kk >= C) & (kk < 2 * C)
                      & (cc >= RES_OFF) & (cc < POS_OFF))
                   | ((kk >= 2 * C) & (cc >= POS_OFF))).astype(jnp.bfloat16)


def _main_body(atom_ref, res_ref, pos_ref, selj_ref, maskr_ref, bsel_ref,
               comb_ref, w2h_ref, b2_ref, out_ref):
    l, r, _ = out_ref.shape
    t = l * r
    vcat = jnp.concatenate(
        [atom_ref[:], res_ref[:] + RES_OFF, pos_ref[:] + POS_OFF],
        axis=1).astype(jnp.float32).astype(jnp.bfloat16)
    ycat = jnp.dot(selj_ref[:], vcat, preferred_element_type=jnp.float32)
    zcat = ycat.astype(jnp.bfloat16) * maskr_ref[:]
    bc = jnp.dot(zcat, bsel_ref[:], preferred_element_type=jnp.float32)
    iota = jax.lax.broadcasted_iota(
        jnp.int32, (t, C), 1).astype(jnp.float32)
    hit = (iota == bc) | (iota == float(B1_ROW))
    oh = hit.astype(jnp.bfloat16)
    h = jnp.dot(oh, comb_ref[:], preferred_element_type=jnp.float32)
    g = h * (1.0 + jax.lax.erf(h * 0.7071067811865476))
    g2 = jnp.dot(g.astype(jnp.bfloat16), w2h_ref[:],
                 preferred_element_type=jnp.float32) + b2_ref[:]
    out_ref[:] = g2.reshape(l, r, C)


@functools.partial(jax.jit, static_argnames=())
def kernel(atom_type, aa_type, aa_pos, atom_table, residue_table, pos_table,
           W1, b1, W2, b2):
    b, l = atom_type.shape
    r = ROW_BLOCK
    t = r * l
    grid = (b // r,)

    combined, w2_half, selj, maskr3, bsel = pl.pallas_call(
        _prep_body,
        out_shape=(jax.ShapeDtypeStruct((C, H), jnp.bfloat16),
                   jax.ShapeDtypeStruct((H, C), jnp.bfloat16),
                   jax.ShapeDtypeStruct((t, l), jnp.bfloat16),
                   jax.ShapeDtypeStruct((t, 3 * r), jnp.bfloat16),
                   jax.ShapeDtypeStruct((H, C), jnp.bfloat16)),
    )(atom_table, residue_table, pos_table, W1, b1.reshape(1, H), W2)

    idx_spec = pl.BlockSpec((l, r), lambda i: (0, i))
    full = lambda shape: pl.BlockSpec(shape, lambda i: (0, 0))

    out = pl.pallas_call(
        _main_body,
        grid=grid,
        in_specs=[
            idx_spec, idx_spec, idx_spec,
            full((t, l)),
            full((t, 3 * r)),
            full((H, C)),
            full((C, H)),
            full((H, C)),
            full((1, C)),
        ],
        out_specs=pl.BlockSpec((l, r, C), lambda i: (0, i, 0)),
        out_shape=jax.ShapeDtypeStruct((l, b, C), jnp.float32),
    )(atom_type.T, aa_type.T, aa_pos.T, selj, maskr3, bsel,
      combined, w2_half, b2.reshape(1, C))

    return out.transpose(1, 0, 2)


# speedup vs baseline: 1.7948x; 1.2125x over previous
import functools

import jax
import jax.numpy as jnp
from jax.experimental import pallas as pl

N_ATOM, N_RES, N_POS = 55, 21, 24
RES_OFF, POS_OFF = 64, 96
B1_ROW = 120
C = 128
H = 3 * C
TOK_BLOCK = 2048


def _prep_body(atom_ref, res_ref, pos_ref, w1_ref, b1_ref, w2_ref, b2_ref,
               combt_ref, w2ht_ref):
    ca = jnp.dot(atom_ref[:], w1_ref[0:C, :],
                 preferred_element_type=jnp.float32)
    cr = jnp.dot(res_ref[:], w1_ref[C:2 * C, :],
                 preferred_element_type=jnp.float32)
    cp = jnp.dot(pos_ref[:], w1_ref[2 * C:3 * C, :],
                 preferred_element_type=jnp.float32)
    z = lambda k: jnp.zeros((k, H), dtype=jnp.float32)
    pieces = [ca, z(RES_OFF - N_ATOM), cr, z(POS_OFF - RES_OFF - N_RES), cp,
              z(B1_ROW - POS_OFF - N_POS), b1_ref[:], z(C - B1_ROW - 1)]
    comb = jnp.concatenate([p for p in pieces if p.shape[0] > 0], axis=0)
    combt_ref[:] = comb.T.astype(jnp.bfloat16)
    w2ht = jnp.concatenate(
        [(w2_ref[:] * 0.5).T, b2_ref[:].T, jnp.zeros((C, 7), jnp.float32)],
        axis=1)
    w2ht_ref[:] = w2ht.astype(jnp.bfloat16)


def _main_body(atom_ref, res_ref, pos_ref, combt_ref, w2ht_ref, out_ref):
    _, tp, _ = out_ref.shape
    va = atom_ref[0]
    vr = res_ref[0] + RES_OFF
    vp = pos_ref[0] + POS_OFF
    iota = jax.lax.broadcasted_iota(jnp.int32, (C, tp), 0)
    hit = (iota == va) | (iota == vr) | (iota == vp) | (iota == B1_ROW)
    oht = hit.astype(jnp.bfloat16)
    ht = jnp.dot(combt_ref[:], oht, preferred_element_type=jnp.float32)
    gt = ht * (1.0 + jax.lax.erf(ht * 0.7071067811865476))
    gt2 = jnp.concatenate(
        [gt.astype(jnp.bfloat16),
         jnp.ones((8, tp), dtype=jnp.bfloat16)], axis=0)
    g2t = jnp.dot(w2ht_ref[:], gt2, preferred_element_type=jnp.float32)
    out_ref[0] = g2t.T


@functools.partial(jax.jit, static_argnames=())
def kernel(atom_type, aa_type, aa_pos, atom_table, residue_table, pos_table,
           W1, b1, W2, b2):
    b, l = atom_type.shape
    tp = TOK_BLOCK
    grid = (l, b // tp)

    combt, w2ht = pl.pallas_call(
        _prep_body,
        out_shape=(jax.ShapeDtypeStruct((H, C), jnp.bfloat16),
                   jax.ShapeDtypeStruct((C, H + 8), jnp.bfloat16)),
    )(atom_table, residue_table, pos_table, W1, b1.reshape(1, H), W2,
      b2.reshape(1, C))

    idx_spec = pl.BlockSpec((1, 1, tp), lambda j, i: (j, 0, i))
    full = lambda shape: pl.BlockSpec(shape, lambda j, i: (0, 0))

    out = pl.pallas_call(
        _main_body,
        grid=grid,
        in_specs=[
            idx_spec, idx_spec, idx_spec,
            full((H, C)),
            full((C, H + 8)),
        ],
        out_specs=pl.BlockSpec((1, tp, C), lambda j, i: (j, i, 0)),
        out_shape=jax.ShapeDtypeStruct((l, b, C), jnp.float32),
    )(atom_type.T.reshape(l, 1, b), aa_type.T.reshape(l, 1, b),
      aa_pos.T.reshape(l, 1, b), combt, w2ht)

    return out.transpose(1, 0, 2)


# speedup vs baseline: 2.0420x; 1.1377x over previous
import functools

import jax
import jax.numpy as jnp
from jax.experimental import pallas as pl

N_ATOM, N_RES, N_POS = 55, 21, 24
RES_OFF, POS_OFF = 64, 96
B1_ROW = 120
C = 128
H = 3 * C
TOK_BLOCK = 4096


def _prep_body(atom_ref, res_ref, pos_ref, w1_ref, b1_ref, w2_ref, b2_ref,
               combt_ref, w2ht_ref):
    ca = jnp.dot(atom_ref[:], w1_ref[0:C, :],
                 preferred_element_type=jnp.float32)
    cr = jnp.dot(res_ref[:], w1_ref[C:2 * C, :],
                 preferred_element_type=jnp.float32)
    cp = jnp.dot(pos_ref[:], w1_ref[2 * C:3 * C, :],
                 preferred_element_type=jnp.float32)
    z = lambda k: jnp.zeros((k, H), dtype=jnp.float32)
    pieces = [ca, z(RES_OFF - N_ATOM), cr, z(POS_OFF - RES_OFF - N_RES), cp,
              z(B1_ROW - POS_OFF - N_POS), b1_ref[:], z(C - B1_ROW - 1)]
    comb = jnp.concatenate([p for p in pieces if p.shape[0] > 0], axis=0)
    combt_ref[:] = comb.T.astype(jnp.bfloat16)
    w2ht = jnp.concatenate(
        [(w2_ref[:] * 0.5).T, b2_ref[:].T, jnp.zeros((C, 7), jnp.float32)],
        axis=1)
    w2ht_ref[:] = w2ht.astype(jnp.bfloat16)


def _main_body(atom_ref, res_ref, pos_ref, combt_ref, w2ht_ref, out_ref):
    _, tp, _ = out_ref.shape
    va = atom_ref[0]
    vr = res_ref[0] + RES_OFF
    vp = pos_ref[0] + POS_OFF
    iota = jax.lax.broadcasted_iota(jnp.int32, (C, tp), 0)
    hit = (iota == va) | (iota == vr) | (iota == vp) | (iota == B1_ROW)
    oht = hit.astype(jnp.bfloat16)
    ht = jnp.dot(combt_ref[:], oht, preferred_element_type=jnp.float32)
    gt = ht * (1.0 + jax.lax.erf(ht * 0.7071067811865476))
    gt2 = jnp.concatenate(
        [gt.astype(jnp.bfloat16),
         jnp.ones((8, tp), dtype=jnp.bfloat16)], axis=0)
    g2t = jnp.dot(w2ht_ref[:], gt2, preferred_element_type=jnp.float32)
    out_ref[0] = g2t.T


@functools.partial(jax.jit, static_argnames=())
def kernel(atom_type, aa_type, aa_pos, atom_table, residue_table, pos_table,
           W1, b1, W2, b2):
    b, l = atom_type.shape
    tp = TOK_BLOCK
    grid = (l, b // tp)

    combt, w2ht = pl.pallas_call(
        _prep_body,
        out_shape=(jax.ShapeDtypeStruct((H, C), jnp.bfloat16),
                   jax.ShapeDtypeStruct((C, H + 8), jnp.bfloat16)),
    )(atom_table, residue_table, pos_table, W1, b1.reshape(1, H), W2,
      b2.reshape(1, C))

    idx_spec = pl.BlockSpec((1, 1, tp), lambda j, i: (j, 0, i))
    full = lambda shape: pl.BlockSpec(shape, lambda j, i: (0, 0))

    out = pl.pallas_call(
        _main_body,
        grid=grid,
        in_specs=[
            idx_spec, idx_spec, idx_spec,
            full((H, C)),
            full((C, H + 8)),
        ],
        out_specs=pl.BlockSpec((1, tp, C), lambda j, i: (j, i, 0)),
        out_shape=jax.ShapeDtypeStruct((l, b, C), jnp.float32),
    )(atom_type.T.reshape(l, 1, b), aa_type.T.reshape(l, 1, b),
      aa_pos.T.reshape(l, 1, b), combt, w2ht)

    return out.transpose(1, 0, 2)
